# baseline (speedup 1.0000x reference)
import functools

import jax
import jax.numpy as jnp
import numpy as np
from jax import lax
from jax.experimental import pallas as pl
from jax.experimental.pallas import tpu as pltpu
from jax.experimental.pallas import tpu_sc as plsc

N = 10000
D = 128
E = 320000

NC = 2
NS = 16
NW = NC * NS
CHUNK = 128
CH = -(-E // (NW * CHUNK))
CH0 = 61
CH1 = 2 * CH - CH0
E_PAD = NS * (CH0 + CH1) * CHUNK

ROWS_PER_TILE = 632
ROWS_LAST = N - (NS - 1) * ROWS_PER_TILE

_PERM = np.empty(D, np.int32)
for _g in range(D // 32):
    for _i in range(16):
        _PERM[32 * _g + 2 * _i] = 32 * _g + _i
        _PERM[32 * _g + 2 * _i + 1] = 32 * _g + 16 + _i


def _mm_body(x_ref, w_ref, b_ref, o_ref):
    o_ref[...] = (lax.dot_general(
        x_ref[...], w_ref[...], (((1,), (1,)), ((), ())),
        preferred_element_type=jnp.float32,
    ) + b_ref[...]).astype(jnp.bfloat16)


def _matmul(x, W, b):
    return pl.pallas_call(
        _mm_body,
        grid=(10,),
        in_specs=[
            pl.BlockSpec((N // 10, D), lambda i: (i, 0)),
            pl.BlockSpec((D, D), lambda i: (0, 0)),
            pl.BlockSpec((1, D), lambda i: (0, 0)),
        ],
        out_specs=pl.BlockSpec((N // 10, D), lambda i: (i, 0)),
        out_shape=jax.ShapeDtypeStruct((N, D), jnp.bfloat16),
    )(x, W, b.reshape(1, D))


def _add_body(p_ref, o_ref):
    o_ref[...] = p_ref[0] + p_ref[1]


def _sum_partials(partials):
    return pl.pallas_call(
        _add_body,
        grid=(10,),
        in_specs=[pl.BlockSpec((2, N // 10, D), lambda i: (0, i, 0))],
        out_specs=pl.BlockSpec((N // 10, D), lambda i: (i, 0)),
        out_shape=jax.ShapeDtypeStruct((N, D), jnp.float32),
    )(partials)


@functools.cache
def _build_sc_scatter():
    mesh = plsc.VectorSubcoreMesh(
        core_axis_name="c", subcore_axis_name="s", num_cores=NC, num_subcores=NS
    )
    return pl.kernel(
        _sc_scatter_body,
        out_type=jax.ShapeDtypeStruct((NC, N, D), jnp.float32),
        mesh=mesh,
        compiler_params=pltpu.CompilerParams(use_tc_tiling_on_sc=False),
        scratch_types=[
            pltpu.VMEM((4, 3, CHUNK), jnp.int32),
            pltpu.VMEM((2, CHUNK, D // 2), jnp.int32),
            pltpu.VMEM((2, CHUNK, D), jnp.float32),
            pltpu.VMEM_SHARED((N, D), jnp.float32),
            pltpu.SemaphoreType.DMA,
            pltpu.SemaphoreType.DMA,
            pltpu.SemaphoreType.DMA,
        ],
    )


def _sc_scatter_body(support_hbm, eidx_hbm, out_hbm,
                     ebuf, rowsb, sbuf, acc, esem, gsem, ssem):
    c = lax.axis_index("c")
    s = lax.axis_index("s")
    wid = s * NC + c

    chc = jnp.where(c == 0, CH0, CH1)
    rec0 = s * (CH0 + CH1) + c * CH0

    pltpu.async_copy(eidx_hbm.at[rec0], ebuf.at[0], esem)

    zvec = jnp.zeros((16,), jnp.float32)

    @plsc.parallel_loop(0, CHUNK, unroll=4)
    def _zrow(i):
        for g in range(D // 16):
            sbuf[0, i, pl.ds(g * 16, 16)] = zvec

    base = s * ROWS_PER_TILE

    def _zero_rows(nrows):
        nfull = nrows // CHUNK
        rem = nrows - nfull * CHUNK
        for k in range(nfull):
            pltpu.sync_copy(sbuf.at[0], acc.at[pl.ds(base + k * CHUNK, CHUNK)])
        if rem:
            pltpu.sync_copy(sbuf.at[0, pl.ds(0, rem)],
                            acc.at[pl.ds(base + nfull * CHUNK, rem)])

    @pl.when(s < NS - 1)
    def _():
        _zero_rows(ROWS_PER_TILE)

    @pl.when(s == NS - 1)
    def _():
        _zero_rows(ROWS_LAST)

    plsc.subcore_barrier()

    pltpu.make_async_copy(eidx_hbm.at[rec0], ebuf.at[0], esem).wait()
    pltpu.async_copy(support_hbm.at[ebuf.at[0, 0]], rowsb.at[0], gsem)
    pltpu.async_copy(eidx_hbm.at[rec0 + 1], ebuf.at[1], esem)

    def _chunk(j, _):
        p = j & 1
        q = j & 3

        pltpu.make_async_copy(support_hbm.at[ebuf.at[q, 0]],
                              rowsb.at[p], gsem).wait()

        @pl.when(j + 1 < chc)
        def _():
            pltpu.make_async_copy(eidx_hbm.at[rec0 + j + 1],
                                  ebuf.at[(j + 1) & 3], esem).wait()

            @pl.when(j >= 1)
            def _():
                pltpu.make_async_copy(sbuf.at[1 - p],
                                      acc.at[ebuf.at[(j - 1) & 3, 1]],
                                      ssem).wait()

            pltpu.async_copy(support_hbm.at[ebuf.at[(j + 1) & 3, 0]],
                             rowsb.at[1 - p], gsem)

        @pl.when(j + 2 < chc)
        def _():
            pltpu.async_copy(eidx_hbm.at[rec0 + j + 2],
                             ebuf.at[(j + 2) & 3], esem)

        @plsc.parallel_loop(0, CHUNK, unroll=4)
        def _scale(e):
            wv = lax.bitcast_convert_type(
                ebuf[q, 2, pl.ds((e // 16) * 16, 16)], jnp.float32)
            w = wv[jnp.full((16,), e % 16, jnp.int32)]
            for g in range(D // 32):
                v = rowsb[p, e, pl.ds(g * 16, 16)]
                a = lax.bitcast_convert_type(v << 16, jnp.float32)
                b2 = lax.bitcast_convert_type(v & jnp.int32(-65536), jnp.float32)
                sbuf[p, e, pl.ds(g * 32, 16)] = a * w
                sbuf[p, e, pl.ds(g * 32 + 16, 16)] = b2 * w

        pltpu.async_copy(sbuf.at[p], acc.at[ebuf.at[q, 1]], ssem, add=True)
        return 0

    lax.fori_loop(0, chc, _chunk, 0)

    pltpu.make_async_copy(sbuf.at[0], acc.at[ebuf.at[0, 1]], ssem).wait()
    pltpu.make_async_copy(sbuf.at[1], acc.at[ebuf.at[1, 1]], ssem).wait()

    plsc.subcore_barrier()

    @pl.when(s < NS - 1)
    def _():
        pltpu.sync_copy(acc.at[pl.ds(base, ROWS_PER_TILE)],
                        out_hbm.at[c, pl.ds(base, ROWS_PER_TILE)])

    @pl.when(s == NS - 1)
    def _():
        pltpu.sync_copy(acc.at[pl.ds(base, ROWS_LAST)],
                        out_hbm.at[c, pl.ds(base, ROWS_LAST)])


def kernel(input, adj_edge_index, adj_edge_weight, W, b):
    perm = jnp.asarray(_PERM)
    support_bf = _matmul(input, W[perm], b[perm])
    support = lax.bitcast_convert_type(
        support_bf.reshape(N, D // 2, 2), jnp.int32)

    row = adj_edge_index[0]
    col = adj_edge_index[1]
    pad = E_PAD - E
    col_p = jnp.pad(col, (0, pad)).reshape(-1, CHUNK)
    row_p = jnp.pad(row, (0, pad)).reshape(-1, CHUNK)
    w_bits = lax.bitcast_convert_type(
        jnp.pad(adj_edge_weight, (0, pad)), jnp.int32).reshape(-1, CHUNK)
    eidx = jnp.stack([col_p, row_p, w_bits], axis=1)

    partials = _build_sc_scatter()(support, eidx)
    return _sum_partials(partials)


# speedup vs baseline: 1.1587x; 1.1587x over previous
import functools

import jax
import jax.numpy as jnp
import numpy as np
from jax import lax
from jax.experimental import pallas as pl
from jax.experimental.pallas import tpu as pltpu
from jax.experimental.pallas import tpu_sc as plsc

N = 10000
D = 128
E = 320000

NC = 2
NS = 16
NW = NC * NS
CHUNK = 128
CH = -(-E // (NW * CHUNK))
CH0 = 97
CH1 = 2 * CH - CH0
E_PAD = NS * (CH0 + CH1) * CHUNK

ROWS_PER_TILE = 632
ROWS_LAST = N - (NS - 1) * ROWS_PER_TILE

_PERM = np.empty(D, np.int32)
for _g in range(D // 32):
    for _i in range(16):
        _PERM[32 * _g + 2 * _i] = 32 * _g + _i
        _PERM[32 * _g + 2 * _i + 1] = 32 * _g + 16 + _i


def _mm_body(x_ref, w_ref, b_ref, o_ref):
    o_ref[...] = (lax.dot_general(
        x_ref[...], w_ref[...], (((1,), (1,)), ((), ())),
        preferred_element_type=jnp.float32,
    ) + b_ref[...]).astype(jnp.bfloat16)


def _matmul(x, W, b):
    return pl.pallas_call(
        _mm_body,
        grid=(10,),
        in_specs=[
            pl.BlockSpec((N // 10, D), lambda i: (i, 0)),
            pl.BlockSpec((D, D), lambda i: (0, 0)),
            pl.BlockSpec((1, D), lambda i: (0, 0)),
        ],
        out_specs=pl.BlockSpec((N // 10, D), lambda i: (i, 0)),
        out_shape=jax.ShapeDtypeStruct((N, D), jnp.bfloat16),
    )(x, W, b.reshape(1, D))


def _add_body(p_ref, o_ref):
    o_ref[...] = p_ref[0] + p_ref[1]


def _sum_partials(partials):
    return pl.pallas_call(
        _add_body,
        grid=(10,),
        in_specs=[pl.BlockSpec((2, N // 10, D), lambda i: (0, i, 0))],
        out_specs=pl.BlockSpec((N // 10, D), lambda i: (i, 0)),
        out_shape=jax.ShapeDtypeStruct((N, D), jnp.float32),
    )(partials)


@functools.cache
def _build_sc_scatter():
    mesh = plsc.VectorSubcoreMesh(
        core_axis_name="c", subcore_axis_name="s", num_cores=NC, num_subcores=NS
    )
    return pl.kernel(
        _sc_scatter_body,
        out_type=jax.ShapeDtypeStruct((NC, N, D), jnp.float32),
        mesh=mesh,
        compiler_params=pltpu.CompilerParams(use_tc_tiling_on_sc=False),
        scratch_types=[
            pltpu.VMEM((4, 3, CHUNK), jnp.int32),
            pltpu.VMEM((2, CHUNK, D // 2), jnp.int32),
            pltpu.VMEM((2, CHUNK, D), jnp.float32),
            pltpu.VMEM_SHARED((N, D), jnp.float32),
            pltpu.SemaphoreType.DMA,
            pltpu.SemaphoreType.DMA,
            pltpu.SemaphoreType.DMA,
        ],
    )


def _sc_scatter_body(support_hbm, eidx_hbm, out_hbm,
                     ebuf, rowsb, sbuf, acc, esem, gsem, ssem):
    c = lax.axis_index("c")
    s = lax.axis_index("s")
    wid = s * NC + c

    chc = jnp.where(c == 0, CH0, CH1)
    rec0 = s * (CH0 + CH1) + c * CH0

    pltpu.async_copy(eidx_hbm.at[rec0], ebuf.at[0], esem)

    zvec = jnp.zeros((16,), jnp.float32)

    @plsc.parallel_loop(0, CHUNK, unroll=4)
    def _zrow(i):
        for g in range(D // 16):
            sbuf[0, i, pl.ds(g * 16, 16)] = zvec

    base = s * ROWS_PER_TILE

    def _zero_rows(nrows):
        nfull = nrows // CHUNK
        rem = nrows - nfull * CHUNK
        for k in range(nfull):
            pltpu.sync_copy(sbuf.at[0], acc.at[pl.ds(base + k * CHUNK, CHUNK)])
        if rem:
            pltpu.sync_copy(sbuf.at[0, pl.ds(0, rem)],
                            acc.at[pl.ds(base + nfull * CHUNK, rem)])

    @pl.when(s < NS - 1)
    def _():
        _zero_rows(ROWS_PER_TILE)

    @pl.when(s == NS - 1)
    def _():
        _zero_rows(ROWS_LAST)

    plsc.subcore_barrier()

    pltpu.make_async_copy(eidx_hbm.at[rec0], ebuf.at[0], esem).wait()
    pltpu.async_copy(support_hbm.at[ebuf.at[0, 0]], rowsb.at[0], gsem)
    pltpu.async_copy(eidx_hbm.at[rec0 + 1], ebuf.at[1], esem)

    def _chunk(j, _):
        p = j & 1
        q = j & 3

        pltpu.make_async_copy(support_hbm.at[ebuf.at[q, 0]],
                              rowsb.at[p], gsem).wait()

        @pl.when(j + 1 < chc)
        def _():
            pltpu.make_async_copy(eidx_hbm.at[rec0 + j + 1],
                                  ebuf.at[(j + 1) & 3], esem).wait()

            @pl.when(j >= 1)
            def _():
                pltpu.make_async_copy(sbuf.at[1 - p],
                                      acc.at[ebuf.at[(j - 1) & 3, 1]],
                                      ssem).wait()

            pltpu.async_copy(support_hbm.at[ebuf.at[(j + 1) & 3, 0]],
                             rowsb.at[1 - p], gsem)

        @pl.when(j + 2 < chc)
        def _():
            pltpu.async_copy(eidx_hbm.at[rec0 + j + 2],
                             ebuf.at[(j + 2) & 3], esem)

        @plsc.parallel_loop(0, CHUNK, unroll=4)
        def _scale(e):
            wv = lax.bitcast_convert_type(
                ebuf[q, 2, pl.ds((e // 16) * 16, 16)], jnp.float32)
            w = wv[jnp.full((16,), e % 16, jnp.int32)]
            for g in range(D // 32):
                v = rowsb[p, e, pl.ds(g * 16, 16)]
                a = lax.bitcast_convert_type(v << 16, jnp.float32)
                b2 = lax.bitcast_convert_type(v & jnp.int32(-65536), jnp.float32)
                sbuf[p, e, pl.ds(g * 32, 16)] = a * w
                sbuf[p, e, pl.ds(g * 32 + 16, 16)] = b2 * w

        pltpu.async_copy(sbuf.at[p], acc.at[ebuf.at[q, 1]], ssem, add=True)
        return 0

    lax.fori_loop(0, chc, _chunk, 0)

    pltpu.make_async_copy(sbuf.at[0], acc.at[ebuf.at[0, 1]], ssem).wait()
    pltpu.make_async_copy(sbuf.at[1], acc.at[ebuf.at[1, 1]], ssem).wait()

    plsc.subcore_barrier()

    @pl.when(s < NS - 1)
    def _():
        pltpu.sync_copy(acc.at[pl.ds(base, ROWS_PER_TILE)],
                        out_hbm.at[c, pl.ds(base, ROWS_PER_TILE)])

    @pl.when(s == NS - 1)
    def _():
        pltpu.sync_copy(acc.at[pl.ds(base, ROWS_LAST)],
                        out_hbm.at[c, pl.ds(base, ROWS_LAST)])


def kernel(input, adj_edge_index, adj_edge_weight, W, b):
    perm = jnp.asarray(_PERM)
    support_bf = _matmul(input, W[perm], b[perm])
    support = lax.bitcast_convert_type(
        support_bf.reshape(N, D // 2, 2), jnp.int32)

    row = adj_edge_index[0]
    col = adj_edge_index[1]
    pad = E_PAD - E
    col_p = jnp.pad(col, (0, pad)).reshape(-1, CHUNK)
    row_p = jnp.pad(row, (0, pad)).reshape(-1, CHUNK)
    w_bits = lax.bitcast_convert_type(
        jnp.pad(adj_edge_weight, (0, pad)), jnp.int32).reshape(-1, CHUNK)
    eidx = jnp.stack([col_p, row_p, w_bits], axis=1)

    partials = _build_sc_scatter()(support, eidx)
    return _sum_partials(partials)


# speedup vs baseline: 1.1948x; 1.0311x over previous
import functools

import jax
import jax.numpy as jnp
import numpy as np
from jax import lax
from jax.experimental import pallas as pl
from jax.experimental.pallas import tpu as pltpu
from jax.experimental.pallas import tpu_sc as plsc

N = 10000
D = 128
E = 320000

NC = 2
NS = 16
NW = NC * NS
CHUNK = 128
CH = -(-E // (NW * CHUNK))
CH0 = 102
CH1 = 2 * CH - CH0
E_PAD = NS * (CH0 + CH1) * CHUNK

ROWS_PER_TILE = 632
ROWS_LAST = N - (NS - 1) * ROWS_PER_TILE

_PERM = np.empty(D, np.int32)
for _g in range(D // 32):
    for _i in range(16):
        _PERM[32 * _g + 2 * _i] = 32 * _g + _i
        _PERM[32 * _g + 2 * _i + 1] = 32 * _g + 16 + _i


def _mm_body(x_ref, w_ref, b_ref, o_ref):
    o_ref[...] = (lax.dot_general(
        x_ref[...], w_ref[...], (((1,), (1,)), ((), ())),
        preferred_element_type=jnp.float32,
    ) + b_ref[...]).astype(jnp.bfloat16)


def _matmul(x, W, b):
    return pl.pallas_call(
        _mm_body,
        grid=(10,),
        in_specs=[
            pl.BlockSpec((N // 10, D), lambda i: (i, 0)),
            pl.BlockSpec((D, D), lambda i: (0, 0)),
            pl.BlockSpec((1, D), lambda i: (0, 0)),
        ],
        out_specs=pl.BlockSpec((N // 10, D), lambda i: (i, 0)),
        out_shape=jax.ShapeDtypeStruct((N, D), jnp.bfloat16),
    )(x, W, b.reshape(1, D))


def _add_body(p_ref, o_ref):
    o_ref[...] = p_ref[0] + p_ref[1]


def _sum_partials(partials):
    return pl.pallas_call(
        _add_body,
        grid=(10,),
        in_specs=[pl.BlockSpec((2, N // 10, D), lambda i: (0, i, 0))],
        out_specs=pl.BlockSpec((N // 10, D), lambda i: (i, 0)),
        out_shape=jax.ShapeDtypeStruct((N, D), jnp.float32),
    )(partials)


@functools.cache
def _build_sc_scatter():
    mesh = plsc.VectorSubcoreMesh(
        core_axis_name="c", subcore_axis_name="s", num_cores=NC, num_subcores=NS
    )
    return pl.kernel(
        _sc_scatter_body,
        out_type=jax.ShapeDtypeStruct((NC, N, D), jnp.float32),
        mesh=mesh,
        compiler_params=pltpu.CompilerParams(use_tc_tiling_on_sc=False),
        scratch_types=[
            pltpu.VMEM((4, 3, CHUNK), jnp.int32),
            pltpu.VMEM((2, CHUNK, D // 2), jnp.int32),
            pltpu.VMEM((2, CHUNK, D), jnp.float32),
            pltpu.VMEM_SHARED((N, D), jnp.float32),
            pltpu.SemaphoreType.DMA,
            pltpu.SemaphoreType.DMA,
            pltpu.SemaphoreType.DMA,
        ],
    )


def _sc_scatter_body(support_hbm, eidx_hbm, out_hbm,
                     ebuf, rowsb, sbuf, acc, esem, gsem, ssem):
    c = lax.axis_index("c")
    s = lax.axis_index("s")
    wid = s * NC + c

    chc = jnp.where(c == 0, CH0, CH1)
    rec0 = s * (CH0 + CH1) + c * CH0

    pltpu.async_copy(eidx_hbm.at[rec0], ebuf.at[0], esem)

    zvec = jnp.zeros((16,), jnp.float32)

    @plsc.parallel_loop(0, CHUNK, unroll=4)
    def _zrow(i):
        for g in range(D // 16):
            sbuf[0, i, pl.ds(g * 16, 16)] = zvec

    base = s * ROWS_PER_TILE

    def _zero_rows(nrows):
        nfull = nrows // CHUNK
        rem = nrows - nfull * CHUNK
        for k in range(nfull):
            pltpu.sync_copy(sbuf.at[0], acc.at[pl.ds(base + k * CHUNK, CHUNK)])
        if rem:
            pltpu.sync_copy(sbuf.at[0, pl.ds(0, rem)],
                            acc.at[pl.ds(base + nfull * CHUNK, rem)])

    @pl.when(s < NS - 1)
    def _():
        _zero_rows(ROWS_PER_TILE)

    @pl.when(s == NS - 1)
    def _():
        _zero_rows(ROWS_LAST)

    plsc.subcore_barrier()

    pltpu.make_async_copy(eidx_hbm.at[rec0], ebuf.at[0], esem).wait()
    pltpu.async_copy(support_hbm.at[ebuf.at[0, 0]], rowsb.at[0], gsem)
    pltpu.async_copy(eidx_hbm.at[rec0 + 1], ebuf.at[1], esem)

    def _chunk(j, _):
        p = j & 1
        q = j & 3

        pltpu.make_async_copy(support_hbm.at[ebuf.at[q, 0]],
                              rowsb.at[p], gsem).wait()

        @pl.when(j + 1 < chc)
        def _():
            pltpu.make_async_copy(eidx_hbm.at[rec0 + j + 1],
                                  ebuf.at[(j + 1) & 3], esem).wait()

            @pl.when(j >= 1)
            def _():
                pltpu.make_async_copy(sbuf.at[1 - p],
                                      acc.at[ebuf.at[(j - 1) & 3, 1]],
                                      ssem).wait()

            pltpu.async_copy(support_hbm.at[ebuf.at[(j + 1) & 3, 0]],
                             rowsb.at[1 - p], gsem)

        @pl.when(j + 2 < chc)
        def _():
            pltpu.async_copy(eidx_hbm.at[rec0 + j + 2],
                             ebuf.at[(j + 2) & 3], esem)

        @plsc.parallel_loop(0, CHUNK, unroll=4)
        def _scale(e):
            wv = lax.bitcast_convert_type(
                ebuf[q, 2, pl.ds((e // 16) * 16, 16)], jnp.float32)
            w = wv[jnp.full((16,), e % 16, jnp.int32)]
            for g in range(D // 32):
                v = rowsb[p, e, pl.ds(g * 16, 16)]
                a = lax.bitcast_convert_type(v << 16, jnp.float32)
                b2 = lax.bitcast_convert_type(v & jnp.int32(-65536), jnp.float32)
                sbuf[p, e, pl.ds(g * 32, 16)] = a * w
                sbuf[p, e, pl.ds(g * 32 + 16, 16)] = b2 * w

        pltpu.async_copy(sbuf.at[p], acc.at[ebuf.at[q, 1]], ssem, add=True)
        return 0

    lax.fori_loop(0, chc, _chunk, 0)

    pltpu.make_async_copy(sbuf.at[0], acc.at[ebuf.at[0, 1]], ssem).wait()
    pltpu.make_async_copy(sbuf.at[1], acc.at[ebuf.at[1, 1]], ssem).wait()

    plsc.subcore_barrier()

    @pl.when(s < NS - 1)
    def _():
        pltpu.sync_copy(acc.at[pl.ds(base, ROWS_PER_TILE)],
                        out_hbm.at[c, pl.ds(base, ROWS_PER_TILE)])

    @pl.when(s == NS - 1)
    def _():
        pltpu.sync_copy(acc.at[pl.ds(base, ROWS_LAST)],
                        out_hbm.at[c, pl.ds(base, ROWS_LAST)])


def kernel(input, adj_edge_index, adj_edge_weight, W, b):
    perm = jnp.asarray(_PERM)
    support_bf = _matmul(input, W[perm], b[perm])
    support = lax.bitcast_convert_type(
        support_bf.reshape(N, D // 2, 2), jnp.int32)

    row = adj_edge_index[0]
    col = adj_edge_index[1]
    pad = E_PAD - E
    col_p = jnp.pad(col, (0, pad)).reshape(-1, CHUNK)
    row_p = jnp.pad(row, (0, pad)).reshape(-1, CHUNK)
    w_bits = lax.bitcast_convert_type(
        jnp.pad(adj_edge_weight, (0, pad)), jnp.int32).reshape(-1, CHUNK)
    eidx = jnp.stack([col_p, row_p, w_bits], axis=1)

    partials = _build_sc_scatter()(support, eidx)
    return _sum_partials(partials)


# speedup vs baseline: 1.3527x; 1.1321x over previous
import functools

import jax
import jax.numpy as jnp
import numpy as np
from jax import lax
from jax.experimental import pallas as pl
from jax.experimental.pallas import tpu as pltpu
from jax.experimental.pallas import tpu_sc as plsc

N = 10000
D = 128
E = 320000

NC = 2
NS = 16
NW = NC * NS
CHUNK = 128
CH = -(-E // (NW * CHUNK))
CH0 = 102
CH1 = 2 * CH - CH0
E_PAD = NS * (CH0 + CH1) * CHUNK

ROWS_PER_TILE = 632
ROWS_LAST = N - (NS - 1) * ROWS_PER_TILE

_PERM = np.empty(D, np.int32)
for _g in range(D // 32):
    for _i in range(16):
        _PERM[16 * _g + _i] = 32 * _g + _i
        _PERM[64 + 16 * _g + _i] = 32 * _g + 16 + _i


def _rne_bf16_bits(f):
    b = pltpu.bitcast(f, jnp.int32)
    lsb = lax.shift_right_logical(b, 16) & 1
    return lax.shift_right_logical(b + 0x7FFF + lsb, 16)


def _mm_body(x_ref, w_ref, b_ref, o_ref):
    res = lax.dot_general(
        x_ref[...], w_ref[...], (((1,), (1,)), ((), ())),
        preferred_element_type=jnp.float32,
    ) + b_ref[...]
    lo = _rne_bf16_bits(res[:, : D // 2])
    hi = _rne_bf16_bits(res[:, D // 2:])
    o_ref[...] = lo | (hi << 16)


def _matmul(x, W, b):
    return pl.pallas_call(
        _mm_body,
        grid=(10,),
        in_specs=[
            pl.BlockSpec((N // 10, D), lambda i: (i, 0)),
            pl.BlockSpec((D, D), lambda i: (0, 0)),
            pl.BlockSpec((1, D), lambda i: (0, 0)),
        ],
        out_specs=pl.BlockSpec((N // 10, D // 2), lambda i: (i, 0)),
        out_shape=jax.ShapeDtypeStruct((N, D // 2), jnp.int32),
    )(x, W, b.reshape(1, D))


def _add_body(p_ref, o_ref):
    o_ref[...] = p_ref[0] + p_ref[1]


def _sum_partials(partials):
    return pl.pallas_call(
        _add_body,
        grid=(10,),
        in_specs=[pl.BlockSpec((2, N // 10, D), lambda i: (0, i, 0))],
        out_specs=pl.BlockSpec((N // 10, D), lambda i: (i, 0)),
        out_shape=jax.ShapeDtypeStruct((N, D), jnp.float32),
    )(partials)


@functools.cache
def _build_sc_scatter():
    mesh = plsc.VectorSubcoreMesh(
        core_axis_name="c", subcore_axis_name="s", num_cores=NC, num_subcores=NS
    )
    return pl.kernel(
        _sc_scatter_body,
        out_type=jax.ShapeDtypeStruct((NC, N, D), jnp.float32),
        mesh=mesh,
        compiler_params=pltpu.CompilerParams(use_tc_tiling_on_sc=False),
        scratch_types=[
            pltpu.VMEM((4, 3, CHUNK), jnp.int32),
            pltpu.VMEM((2, CHUNK, D // 2), jnp.int32),
            pltpu.VMEM((2, CHUNK, D), jnp.float32),
            pltpu.VMEM_SHARED((N, D), jnp.float32),
            pltpu.SemaphoreType.DMA,
            pltpu.SemaphoreType.DMA,
            pltpu.SemaphoreType.DMA,
        ],
    )


def _sc_scatter_body(support_hbm, eidx_hbm, out_hbm,
                     ebuf, rowsb, sbuf, acc, esem, gsem, ssem):
    c = lax.axis_index("c")
    s = lax.axis_index("s")
    wid = s * NC + c

    chc = jnp.where(c == 0, CH0, CH1)
    rec0 = s * (CH0 + CH1) + c * CH0

    pltpu.async_copy(eidx_hbm.at[rec0], ebuf.at[0], esem)

    zvec = jnp.zeros((16,), jnp.float32)

    @plsc.parallel_loop(0, CHUNK, unroll=4)
    def _zrow(i):
        for g in range(D // 16):
            sbuf[0, i, pl.ds(g * 16, 16)] = zvec

    base = s * ROWS_PER_TILE

    def _zero_rows(nrows):
        nfull = nrows // CHUNK
        rem = nrows - nfull * CHUNK
        for k in range(nfull):
            pltpu.sync_copy(sbuf.at[0], acc.at[pl.ds(base + k * CHUNK, CHUNK)])
        if rem:
            pltpu.sync_copy(sbuf.at[0, pl.ds(0, rem)],
                            acc.at[pl.ds(base + nfull * CHUNK, rem)])

    @pl.when(s < NS - 1)
    def _():
        _zero_rows(ROWS_PER_TILE)

    @pl.when(s == NS - 1)
    def _():
        _zero_rows(ROWS_LAST)

    plsc.subcore_barrier()

    pltpu.make_async_copy(eidx_hbm.at[rec0], ebuf.at[0], esem).wait()
    pltpu.async_copy(support_hbm.at[ebuf.at[0, 0]], rowsb.at[0], gsem)
    pltpu.async_copy(eidx_hbm.at[rec0 + 1], ebuf.at[1], esem)

    def _chunk(j, _):
        p = j & 1
        q = j & 3

        pltpu.make_async_copy(support_hbm.at[ebuf.at[q, 0]],
                              rowsb.at[p], gsem).wait()

        @pl.when(j + 1 < chc)
        def _():
            pltpu.make_async_copy(eidx_hbm.at[rec0 + j + 1],
                                  ebuf.at[(j + 1) & 3], esem).wait()

            @pl.when(j >= 1)
            def _():
                pltpu.make_async_copy(sbuf.at[1 - p],
                                      acc.at[ebuf.at[(j - 1) & 3, 1]],
                                      ssem).wait()

            pltpu.async_copy(support_hbm.at[ebuf.at[(j + 1) & 3, 0]],
                             rowsb.at[1 - p], gsem)

        @pl.when(j + 2 < chc)
        def _():
            pltpu.async_copy(eidx_hbm.at[rec0 + j + 2],
                             ebuf.at[(j + 2) & 3], esem)

        @plsc.parallel_loop(0, CHUNK, unroll=4)
        def _scale(e):
            wv = lax.bitcast_convert_type(
                ebuf[q, 2, pl.ds((e // 16) * 16, 16)], jnp.float32)
            w = wv[jnp.full((16,), e % 16, jnp.int32)]
            for g in range(D // 32):
                v = rowsb[p, e, pl.ds(g * 16, 16)]
                a = lax.bitcast_convert_type(v << 16, jnp.float32)
                b2 = lax.bitcast_convert_type(v & jnp.int32(-65536), jnp.float32)
                sbuf[p, e, pl.ds(g * 32, 16)] = a * w
                sbuf[p, e, pl.ds(g * 32 + 16, 16)] = b2 * w

        pltpu.async_copy(sbuf.at[p], acc.at[ebuf.at[q, 1]], ssem, add=True)
        return 0

    lax.fori_loop(0, chc, _chunk, 0)

    pltpu.make_async_copy(sbuf.at[0], acc.at[ebuf.at[0, 1]], ssem).wait()
    pltpu.make_async_copy(sbuf.at[1], acc.at[ebuf.at[1, 1]], ssem).wait()

    plsc.subcore_barrier()

    @pl.when(s < NS - 1)
    def _():
        pltpu.sync_copy(acc.at[pl.ds(base, ROWS_PER_TILE)],
                        out_hbm.at[c, pl.ds(base, ROWS_PER_TILE)])

    @pl.when(s == NS - 1)
    def _():
        pltpu.sync_copy(acc.at[pl.ds(base, ROWS_LAST)],
                        out_hbm.at[c, pl.ds(base, ROWS_LAST)])


def kernel(input, adj_edge_index, adj_edge_weight, W, b):
    perm = jnp.asarray(_PERM)
    support = _matmul(input, W[perm], b[perm])

    row = adj_edge_index[0]
    col = adj_edge_index[1]
    pad = E_PAD - E
    col_p = jnp.pad(col, (0, pad)).reshape(-1, CHUNK)
    row_p = jnp.pad(row, (0, pad)).reshape(-1, CHUNK)
    w_bits = lax.bitcast_convert_type(
        jnp.pad(adj_edge_weight, (0, pad)), jnp.int32).reshape(-1, CHUNK)
    eidx = jnp.stack([col_p, row_p, w_bits], axis=1)

    partials = _build_sc_scatter()(support, eidx)
    return _sum_partials(partials)
